# baseline (speedup 1.0000x reference)
import functools

import jax
import jax.numpy as jnp
from jax import lax
from jax.experimental import pallas as pl
from jax.experimental.pallas import tpu as pltpu
from jax.experimental.pallas import tpu_sc as plsc

NC = 2
NS = 16
HALF = 128
KC = 40
PREC = None
NB = 5
F32 = jnp.float32


def _mesh():
    return plsc.VectorSubcoreMesh(core_axis_name="c", subcore_axis_name="s",
                                  num_cores=NC, num_subcores=NS)


def _make_seg_sum(n_dst, n_edges):
    e_per_sub = n_edges // NS
    n_chunks = e_per_sub // KC
    n_groups = n_chunks // NB
    assert n_chunks % NB == 0
    slab = (n_dst // NS) & ~7
    tail = n_dst - NS * slab

    @functools.partial(
        pl.kernel,
        out_type=jax.ShapeDtypeStruct((NC, n_dst, HALF), F32),
        mesh=_mesh(),
        scratch_types=[
            pltpu.VMEM((2, NB, KC), jnp.int32),
            pltpu.VMEM((2, NB, KC), jnp.int32),
            pltpu.VMEM((NB, KC, HALF), F32),
            pltpu.VMEM_SHARED((n_dst, HALF), F32),
            pltpu.SemaphoreType.DMA((2,)),
            pltpu.SemaphoreType.DMA((2,)),
            pltpu.SemaphoreType.DMA((NB,)),
            pltpu.SemaphoreType.DMA((NB,)),
        ],
    )
    def seg(x_lo, x_hi, src, dst, zrs, out, src_b, dst_b, rows_v, acc,
            s_isem, d_isem, gsem, ssem):
        c = lax.axis_index("c")
        s = lax.axis_index("s")
        r0 = s * slab

        def fetch_idx(g, p):
            pltpu.async_copy(src.at[s, g], src_b.at[p], s_isem.at[p])
            pltpu.async_copy(dst.at[s, g], dst_b.at[p], d_isem.at[p])

        def wait_idx(g, p):
            pltpu.make_async_copy(src.at[s, g], src_b.at[p],
                                  s_isem.at[p]).wait()
            pltpu.make_async_copy(dst.at[s, g], dst_b.at[p],
                                  d_isem.at[p]).wait()

        fetch_idx(0, 0)
        pltpu.sync_copy(zrs.at[pl.ds(0, slab)], acc.at[pl.ds(r0, slab)])
        if tail:
            @pl.when(s == NS - 1)
            def _zt():
                pltpu.sync_copy(zrs.at[pl.ds(0, tail)],
                                acc.at[pl.ds(NS * slab, tail)])
        plsc.subcore_barrier()

        def run(x_tab):
            def group(g, carry):
                p = lax.rem(g, 2)
                wait_idx(g, p)
                descs = []
                for b in range(NB):
                    @pl.when(g > 0)
                    def _drain(b=b):
                        pltpu.make_async_copy(
                            rows_v.at[b], acc.at[dst_b.at[1 - p, b]],
                            ssem.at[b]).wait()
                    descs.append(pltpu.async_copy(
                        x_tab.at[src_b.at[p, b]], rows_v.at[b], gsem.at[b]))

                @pl.when(g + 1 < n_groups)
                def _pref():
                    fetch_idx(g + 1, 1 - p)
                for b in range(NB):
                    descs[b].wait()
                    pltpu.async_copy(rows_v.at[b], acc.at[dst_b.at[p, b]],
                                     ssem.at[b], add=True)
                return carry
            lax.fori_loop(0, n_groups, group, 0)
            pl_ = (n_groups - 1) % 2
            for b in range(NB):
                pltpu.make_async_copy(rows_v.at[b], acc.at[dst_b.at[pl_, b]],
                                      ssem.at[b]).wait()

        @pl.when(c == 0)
        def _lo():
            run(x_lo)

        @pl.when(c == 1)
        def _hi():
            run(x_hi)

        plsc.subcore_barrier()
        pltpu.sync_copy(acc.at[pl.ds(r0, slab)],
                        out.at[c, pl.ds(r0, slab), :])
        if tail:
            @pl.when(s == NS - 1)
            def _ft():
                pltpu.sync_copy(acc.at[pl.ds(NS * slab, tail)],
                                out.at[c, pl.ds(NS * slab, tail), :])

    return seg


def _make_counts(n_dst, n_edges):
    e_per_sub = n_edges // NS
    n_chunks = e_per_sub // KC
    slab = (n_dst // NS) & ~7
    tail = n_dst - NS * slab

    n_groups = n_chunks // NB

    @functools.partial(
        pl.kernel,
        out_type=jax.ShapeDtypeStruct((NC, n_dst, HALF), F32),
        mesh=_mesh(),
        scratch_types=[
            pltpu.VMEM((2, NB, KC), jnp.int32),
            pltpu.VMEM((KC, HALF), F32),
            pltpu.VMEM_SHARED((n_dst, HALF), F32),
            pltpu.SemaphoreType.DMA((2,)),
            pltpu.SemaphoreType.DMA((NB,)),
        ],
    )
    def cnt(dst_a, dst_b, zrs, ones, out, idx_b, ones_v, acc, isem, ssem):
        c = lax.axis_index("c")
        s = lax.axis_index("s")
        r0 = s * slab
        pltpu.sync_copy(zrs.at[pl.ds(0, slab)], acc.at[pl.ds(r0, slab)])
        if tail:
            @pl.when(s == NS - 1)
            def _zt():
                pltpu.sync_copy(zrs.at[pl.ds(0, tail)],
                                acc.at[pl.ds(NS * slab, tail)])
        pltpu.sync_copy(ones, ones_v)

        def run(dref):
            def fetch_idx(g, p):
                pltpu.async_copy(dref.at[s, g], idx_b.at[p], isem.at[p])

            def wait_idx(g, p):
                pltpu.make_async_copy(dref.at[s, g], idx_b.at[p],
                                      isem.at[p]).wait()

            fetch_idx(0, 0)
            plsc.subcore_barrier()

            def group(g, carry):
                p = lax.rem(g, 2)
                wait_idx(g, p)
                for b in range(NB):
                    @pl.when(g > 0)
                    def _drain(b=b):
                        pltpu.make_async_copy(
                            ones_v, acc.at[idx_b.at[1 - p, b]],
                            ssem.at[b]).wait()
                    pltpu.async_copy(ones_v, acc.at[idx_b.at[p, b]],
                                     ssem.at[b], add=True)

                @pl.when(g + 1 < n_groups)
                def _pref():
                    fetch_idx(g + 1, 1 - p)
                return carry
            lax.fori_loop(0, n_groups, group, 0)
            pl_ = (n_groups - 1) % 2
            for b in range(NB):
                pltpu.make_async_copy(ones_v, acc.at[idx_b.at[pl_, b]],
                                      ssem.at[b]).wait()

        @pl.when(c == 0)
        def _a():
            run(dst_a)

        @pl.when(c == 1)
        def _b():
            run(dst_b)

        plsc.subcore_barrier()
        pltpu.sync_copy(acc.at[pl.ds(r0, slab)],
                        out.at[c, pl.ds(r0, slab), :])
        if tail:
            @pl.when(s == NS - 1)
            def _ft():
                pltpu.sync_copy(acc.at[pl.ds(NS * slab, tail)],
                                out.at[c, pl.ds(NS * slab, tail), :])

    return cnt


def _make_pair_gather(n_lab, width):
    n_chunks = n_lab // KC
    nw = NC * NS
    per_w = -(-n_chunks // nw)

    @functools.partial(
        pl.kernel,
        out_type=(jax.ShapeDtypeStruct((n_lab, width), F32),
                  jax.ShapeDtypeStruct((n_lab, width), F32)),
        mesh=_mesh(),
        scratch_types=[
            pltpu.VMEM((2, 1, KC), jnp.int32),
            pltpu.VMEM((2, 1, KC), jnp.int32),
            pltpu.VMEM((2, KC, width), F32),
            pltpu.VMEM((2, KC, width), F32),
            pltpu.SemaphoreType.DMA((2,)),
            pltpu.SemaphoreType.DMA((2,)),
            pltpu.SemaphoreType.DMA((2,)),
            pltpu.SemaphoreType.DMA((2,)),
            pltpu.SemaphoreType.DMA((2,)),
            pltpu.SemaphoreType.DMA((2,)),
        ],
    )
    def gat(tab_a, tab_b, idx_a, idx_b, zs, zd, ia, ib, ra, rb,
            iasem, ibsem, gasem, gbsem, wasem, wbsem):
        c = lax.axis_index("c")
        s = lax.axis_index("s")
        w = s * NC + c

        def fetch_idx(k, p):
            pltpu.async_copy(idx_a.at[k], ia.at[p], iasem.at[p])
            pltpu.async_copy(idx_b.at[k], ib.at[p], ibsem.at[p])

        @pl.when(w < n_chunks)
        def _p0():
            fetch_idx(w, 0)

        def body(j, carry):
            p = lax.rem(j, 2)
            k = w + nw * j

            @pl.when(k < n_chunks)
            def _work():
                pltpu.make_async_copy(idx_a.at[k], ia.at[p],
                                      iasem.at[p]).wait()
                pltpu.make_async_copy(idx_b.at[k], ib.at[p],
                                      ibsem.at[p]).wait()

                @pl.when(j >= 2)
                def _drain():
                    pltpu.make_async_copy(ra.at[p], zs.at[pl.ds(0, KC), :],
                                          wasem.at[p]).wait()
                    pltpu.make_async_copy(rb.at[p], zd.at[pl.ds(0, KC), :],
                                          wbsem.at[p]).wait()
                ga = pltpu.async_copy(tab_a.at[ia.at[p, 0]], ra.at[p],
                                      gasem.at[p])
                gb = pltpu.async_copy(tab_b.at[ib.at[p, 0]], rb.at[p],
                                      gbsem.at[p])

                @pl.when(k + nw < n_chunks)
                def _pref():
                    fetch_idx(k + nw, 1 - p)
                ga.wait()
                gb.wait()
                base = k * KC
                pltpu.async_copy(ra.at[p], zs.at[pl.ds(base, KC), :],
                                 wasem.at[p])
                pltpu.async_copy(rb.at[p], zd.at[pl.ds(base, KC), :],
                                 wbsem.at[p])
            return carry

        lax.fori_loop(0, per_w, body, 0)
        nu = lax.div(n_chunks - 1 - w, nw) + 1

        @pl.when(w < n_chunks)
        def _ep():
            for t in (2, 1):
                @pl.when(nu >= t)
                def _dr(t=t):
                    p = lax.rem(nu - t, 2)
                    pltpu.make_async_copy(ra.at[p], zs.at[pl.ds(0, KC), :],
                                          wasem.at[p]).wait()
                    pltpu.make_async_copy(rb.at[p], zd.at[pl.ds(0, KC), :],
                                          wbsem.at[p]).wait()

    return gat


def _layer(sum_m, sum_u, cnts, xm_lo, xm_hi, xu_lo, xu_hi,
           wml, bm, wmr, wul, bu, wur, relu, split):
    n = sum_m.shape[1]
    h = wml.shape[0]
    br = 1000
    grid = (n // br,)

    def body(sm, su, cn, xml, xmh, xul, xuh, rwml, rbm, rwmr, rwul, rbu,
             rwur, *outs):
        cm = jnp.maximum(cn[0, :, 0:1], 1.0)
        cu = jnp.maximum(cn[1, :, 0:1], 1.0)

        def sage(slo, shi, cc, xlo, xhi, wl, b, wr):
            m = (jnp.dot(slo / cc, wl[:HALF, :], precision=PREC,
                         preferred_element_type=F32)
                 + jnp.dot(shi / cc, wl[HALF:, :], precision=PREC,
                           preferred_element_type=F32))
            x = (jnp.dot(xlo, wr[:HALF, :], precision=PREC,
                         preferred_element_type=F32)
                 + jnp.dot(xhi, wr[HALF:, :], precision=PREC,
                           preferred_element_type=F32))
            return m + x + b

        hm = sage(sm[0], sm[1], cm, xml[...], xmh[...], rwml[...], rbm[...],
                  rwmr[...])
        hu = sage(su[0], su[1], cu, xul[...], xuh[...], rwul[...], rbu[...],
                  rwur[...])
        if relu:
            hm = jnp.maximum(hm, 0.0)
            hu = jnp.maximum(hu, 0.0)
        if split:
            outs[0][...] = hm[:, :HALF]
            outs[1][...] = hm[:, HALF:]
            outs[2][...] = hu[:, :HALF]
            outs[3][...] = hu[:, HALF:]
        else:
            outs[0][...] = hm
            outs[1][...] = hu

    half_spec = pl.BlockSpec((br, HALF), lambda i: (i, 0))
    w_spec = pl.BlockSpec((h, h), lambda i: (0, 0))
    b_spec = pl.BlockSpec((1, h), lambda i: (0, 0))
    in_specs = [
        pl.BlockSpec((NC, br, HALF), lambda i: (0, i, 0)),
        pl.BlockSpec((NC, br, HALF), lambda i: (0, i, 0)),
        pl.BlockSpec((NC, br, HALF), lambda i: (0, i, 0)),
        half_spec, half_spec, half_spec, half_spec,
        w_spec, b_spec, w_spec, w_spec, b_spec, w_spec,
    ]
    if split:
        out_shape = [jax.ShapeDtypeStruct((n, HALF), F32)] * 4
        out_specs = [pl.BlockSpec((br, HALF), lambda i: (i, 0))] * 4
    else:
        out_shape = [jax.ShapeDtypeStruct((n, h), F32)] * 2
        out_specs = [pl.BlockSpec((br, h), lambda i: (i, 0))] * 2

    return pl.pallas_call(
        body, grid=grid, in_specs=in_specs, out_specs=out_specs,
        out_shape=out_shape,
    )(sum_m, sum_u, cnts, xm_lo, xm_hi, xu_lo, xu_hi,
      wml, bm, wmr, wul, bu, wur)


def _scores(zs, zd, wd2, r):
    n_lab, h = zs.shape
    bl = 1000
    grid = (n_lab // bl,)

    def body(a, b, w, o):
        av = a[...]
        bv = b[...]
        ones = jnp.zeros((h, 1), F32) + 1.0
        t = jnp.dot(av, w[...], precision=PREC, preferred_element_type=F32)
        cols = []
        for j in range(r):
            cols.append(jnp.dot(t[:, j * h:(j + 1) * h] * bv, ones,
                                precision=PREC, preferred_element_type=F32))
        o[...] = jnp.concatenate(cols, axis=1)

    return pl.pallas_call(
        body, grid=grid,
        in_specs=[pl.BlockSpec((bl, h), lambda i: (i, 0)),
                  pl.BlockSpec((bl, h), lambda i: (i, 0)),
                  pl.BlockSpec((h, r * h), lambda i: (0, 0))],
        out_specs=pl.BlockSpec((bl, r), lambda i: (i, 0)),
        out_shape=jax.ShapeDtypeStruct((n_lab, r), F32),
    )(zs, zd, wd2)


def _log_softmax0(scores):
    def body(x_ref, o_ref):
        x = x_ref[...]
        m = jnp.max(x, axis=0, keepdims=True)
        e = jnp.exp(x - m)
        ssum = jnp.sum(e, axis=0, keepdims=True)
        o_ref[...] = x - m - jnp.log(ssum)

    return pl.pallas_call(
        body, out_shape=jax.ShapeDtypeStruct(scores.shape, F32),
    )(scores)


def kernel(x_user, x_movie, edge_index_um, edge_index_mu, edge_label_index,
           W1_um_l, b1_um, W1_um_r, W1_mu_l, b1_mu, W1_mu_r,
           W2_um_l, b2_um, W2_um_r, W2_mu_l, b2_mu, W2_mu_r, W_dec):
    n, h = x_user.shape
    e = edge_index_um.shape[1]
    n_lab = edge_label_index.shape[1]
    r = W_dec.shape[0]

    su, du = edge_index_um[0], edge_index_um[1]
    sm, dm = edge_index_mu[0], edge_index_mu[1]
    su3 = su.reshape(NS, -1, NB, KC)
    du3 = du.reshape(NS, -1, NB, KC)
    sm3 = sm.reshape(NS, -1, NB, KC)
    dm3 = dm.reshape(NS, -1, NB, KC)
    el0, el1 = edge_label_index[0], edge_label_index[1]
    xu_lo, xu_hi = x_user[:, :HALF], x_user[:, HALF:]
    xm_lo, xm_hi = x_movie[:, :HALF], x_movie[:, HALF:]

    zeros_h = jnp.zeros((n // NS, HALF), F32)
    ones_h = jnp.ones((KC, HALF), F32)

    cnts = _make_counts(n, e)(du3, dm3, zeros_h, ones_h)
    seg = _make_seg_sum(n, e)
    sum_m1 = seg(xu_lo, xu_hi, su3, du3, zeros_h)
    sum_u1 = seg(xm_lo, xm_hi, sm3, dm3, zeros_h)

    hm_lo, hm_hi, hu_lo, hu_hi = _layer(
        sum_m1, sum_u1, cnts, xm_lo, xm_hi, xu_lo, xu_hi,
        W1_um_l, b1_um.reshape(1, -1), W1_um_r,
        W1_mu_l, b1_mu.reshape(1, -1), W1_mu_r, relu=True, split=True)

    sum_m2 = seg(hu_lo, hu_hi, su3, du3, zeros_h)
    sum_u2 = seg(hm_lo, hm_hi, sm3, dm3, zeros_h)

    z_movie, z_user = _layer(
        sum_m2, sum_u2, cnts, hm_lo, hm_hi, hu_lo, hu_hi,
        W2_um_l, b2_um.reshape(1, -1), W2_um_r,
        W2_mu_l, b2_mu.reshape(1, -1), W2_mu_r, relu=False, split=False)

    zs, zd = _make_pair_gather(n_lab, h)(z_user, z_movie,
                                         el0.reshape(-1, 1, KC),
                                         el1.reshape(-1, 1, KC))

    wd2 = jnp.transpose(W_dec, (1, 0, 2)).reshape(h, r * h)
    sc = _scores(zs, zd, wd2, r)
    return _log_softmax0(sc)


# speedup vs baseline: 1.0045x; 1.0045x over previous
import functools

import jax
import jax.numpy as jnp
from jax import lax
from jax.experimental import pallas as pl
from jax.experimental.pallas import tpu as pltpu
from jax.experimental.pallas import tpu_sc as plsc

NC = 2
NS = 16
HALF = 128
KC = 40
PREC = None
NB = 5
F32 = jnp.float32


def _mesh():
    return plsc.VectorSubcoreMesh(core_axis_name="c", subcore_axis_name="s",
                                  num_cores=NC, num_subcores=NS)


def _make_seg_sum(n_dst, n_edges):
    e_per_sub = n_edges // NS
    n_chunks = e_per_sub // KC
    n_groups = n_chunks // NB
    assert n_chunks % NB == 0
    slab = (n_dst // NS) & ~7
    tail = n_dst - NS * slab

    @functools.partial(
        pl.kernel,
        out_type=(jax.ShapeDtypeStruct((NC, n_dst, HALF), F32),
                  jax.ShapeDtypeStruct((NC, n_dst, HALF), F32)),
        mesh=_mesh(),
        scratch_types=[
            pltpu.VMEM((2, NB, KC), jnp.int32),
            pltpu.VMEM((2, NB, KC), jnp.int32),
            pltpu.VMEM((NB, KC, HALF), F32),
            pltpu.VMEM_SHARED((n_dst, HALF), F32),
            pltpu.SemaphoreType.DMA((2,)),
            pltpu.SemaphoreType.DMA((2,)),
            pltpu.SemaphoreType.DMA((NB,)),
            pltpu.SemaphoreType.DMA((NB,)),
        ],
    )
    def seg(xa_lo, xa_hi, src_a, dst_a, xb_lo, xb_hi, src_b_, dst_b_, zrs,
            out_a, out_b, src_b, dst_b, rows_v, acc, s_isem, d_isem, gsem,
            ssem):
        c = lax.axis_index("c")
        s = lax.axis_index("s")
        r0 = s * slab

        def fetch_idx(src, dst, g, p):
            pltpu.async_copy(src.at[s, g], src_b.at[p], s_isem.at[p])
            pltpu.async_copy(dst.at[s, g], dst_b.at[p], d_isem.at[p])

        def wait_idx(src, dst, g, p):
            pltpu.make_async_copy(src.at[s, g], src_b.at[p],
                                  s_isem.at[p]).wait()
            pltpu.make_async_copy(dst.at[s, g], dst_b.at[p],
                                  d_isem.at[p]).wait()

        def zero_acc():
            pltpu.sync_copy(zrs.at[pl.ds(0, slab)], acc.at[pl.ds(r0, slab)])
            if tail:
                @pl.when(s == NS - 1)
                def _zt():
                    pltpu.sync_copy(zrs.at[pl.ds(0, tail)],
                                    acc.at[pl.ds(NS * slab, tail)])

        def flush(out):
            pltpu.sync_copy(acc.at[pl.ds(r0, slab)],
                            out.at[c, pl.ds(r0, slab), :])
            if tail:
                @pl.when(s == NS - 1)
                def _ft():
                    pltpu.sync_copy(acc.at[pl.ds(NS * slab, tail)],
                                    out.at[c, pl.ds(NS * slab, tail), :])

        def run(x_tab, src, dst):
            def group(g, carry):
                p = lax.rem(g, 2)
                wait_idx(src, dst, g, p)
                descs = []
                for b in range(NB):
                    @pl.when(g > 0)
                    def _drain(b=b):
                        pltpu.make_async_copy(
                            rows_v.at[b], acc.at[dst_b.at[1 - p, b]],
                            ssem.at[b]).wait()
                    descs.append(pltpu.async_copy(
                        x_tab.at[src_b.at[p, b]], rows_v.at[b], gsem.at[b]))

                @pl.when(g + 1 < n_groups)
                def _pref():
                    fetch_idx(src, dst, g + 1, 1 - p)
                for b in range(NB):
                    descs[b].wait()
                    pltpu.async_copy(rows_v.at[b], acc.at[dst_b.at[p, b]],
                                     ssem.at[b], add=True)
                return carry
            lax.fori_loop(0, n_groups, group, 0)
            pl_ = (n_groups - 1) % 2
            for b in range(NB):
                pltpu.make_async_copy(rows_v.at[b], acc.at[dst_b.at[pl_, b]],
                                      ssem.at[b]).wait()

        def phase(xlo, xhi, src, dst, out):
            zero_acc()
            fetch_idx(src, dst, 0, 0)
            plsc.subcore_barrier()

            @pl.when(c == 0)
            def _lo():
                run(xlo, src, dst)

            @pl.when(c == 1)
            def _hi():
                run(xhi, src, dst)

            plsc.subcore_barrier()
            flush(out)

        phase(xa_lo, xa_hi, src_a, dst_a, out_a)
        plsc.subcore_barrier()
        phase(xb_lo, xb_hi, src_b_, dst_b_, out_b)

    return seg


def _make_counts(n_dst, n_edges):
    e_per_sub = n_edges // NS
    n_chunks = e_per_sub // KC
    slab = (n_dst // NS) & ~7
    tail = n_dst - NS * slab

    n_groups = n_chunks // NB

    @functools.partial(
        pl.kernel,
        out_type=jax.ShapeDtypeStruct((NC, n_dst, HALF), F32),
        mesh=_mesh(),
        scratch_types=[
            pltpu.VMEM((2, NB, KC), jnp.int32),
            pltpu.VMEM((KC, HALF), F32),
            pltpu.VMEM_SHARED((n_dst, HALF), F32),
            pltpu.SemaphoreType.DMA((2,)),
            pltpu.SemaphoreType.DMA((NB,)),
        ],
    )
    def cnt(dst_a, dst_b, zrs, ones, out, idx_b, ones_v, acc, isem, ssem):
        c = lax.axis_index("c")
        s = lax.axis_index("s")
        r0 = s * slab
        pltpu.sync_copy(zrs.at[pl.ds(0, slab)], acc.at[pl.ds(r0, slab)])
        if tail:
            @pl.when(s == NS - 1)
            def _zt():
                pltpu.sync_copy(zrs.at[pl.ds(0, tail)],
                                acc.at[pl.ds(NS * slab, tail)])
        pltpu.sync_copy(ones, ones_v)

        def run(dref):
            def fetch_idx(g, p):
                pltpu.async_copy(dref.at[s, g], idx_b.at[p], isem.at[p])

            def wait_idx(g, p):
                pltpu.make_async_copy(dref.at[s, g], idx_b.at[p],
                                      isem.at[p]).wait()

            fetch_idx(0, 0)
            plsc.subcore_barrier()

            def group(g, carry):
                p = lax.rem(g, 2)
                wait_idx(g, p)
                for b in range(NB):
                    @pl.when(g > 0)
                    def _drain(b=b):
                        pltpu.make_async_copy(
                            ones_v, acc.at[idx_b.at[1 - p, b]],
                            ssem.at[b]).wait()
                    pltpu.async_copy(ones_v, acc.at[idx_b.at[p, b]],
                                     ssem.at[b], add=True)

                @pl.when(g + 1 < n_groups)
                def _pref():
                    fetch_idx(g + 1, 1 - p)
                return carry
            lax.fori_loop(0, n_groups, group, 0)
            pl_ = (n_groups - 1) % 2
            for b in range(NB):
                pltpu.make_async_copy(ones_v, acc.at[idx_b.at[pl_, b]],
                                      ssem.at[b]).wait()

        @pl.when(c == 0)
        def _a():
            run(dst_a)

        @pl.when(c == 1)
        def _b():
            run(dst_b)

        plsc.subcore_barrier()
        pltpu.sync_copy(acc.at[pl.ds(r0, slab)],
                        out.at[c, pl.ds(r0, slab), :])
        if tail:
            @pl.when(s == NS - 1)
            def _ft():
                pltpu.sync_copy(acc.at[pl.ds(NS * slab, tail)],
                                out.at[c, pl.ds(NS * slab, tail), :])

    return cnt


def _make_pair_gather(n_lab, width):
    n_chunks = n_lab // KC
    nw = NC * NS
    per_w = -(-n_chunks // nw)

    @functools.partial(
        pl.kernel,
        out_type=(jax.ShapeDtypeStruct((n_lab, width), F32),
                  jax.ShapeDtypeStruct((n_lab, width), F32)),
        mesh=_mesh(),
        scratch_types=[
            pltpu.VMEM((2, 1, KC), jnp.int32),
            pltpu.VMEM((2, 1, KC), jnp.int32),
            pltpu.VMEM((2, KC, width), F32),
            pltpu.VMEM((2, KC, width), F32),
            pltpu.SemaphoreType.DMA((2,)),
            pltpu.SemaphoreType.DMA((2,)),
            pltpu.SemaphoreType.DMA((2,)),
            pltpu.SemaphoreType.DMA((2,)),
            pltpu.SemaphoreType.DMA((2,)),
            pltpu.SemaphoreType.DMA((2,)),
        ],
    )
    def gat(tab_a, tab_b, idx_a, idx_b, zs, zd, ia, ib, ra, rb,
            iasem, ibsem, gasem, gbsem, wasem, wbsem):
        c = lax.axis_index("c")
        s = lax.axis_index("s")
        w = s * NC + c

        def fetch_idx(k, p):
            pltpu.async_copy(idx_a.at[k], ia.at[p], iasem.at[p])
            pltpu.async_copy(idx_b.at[k], ib.at[p], ibsem.at[p])

        @pl.when(w < n_chunks)
        def _p0():
            fetch_idx(w, 0)

        def body(j, carry):
            p = lax.rem(j, 2)
            k = w + nw * j

            @pl.when(k < n_chunks)
            def _work():
                pltpu.make_async_copy(idx_a.at[k], ia.at[p],
                                      iasem.at[p]).wait()
                pltpu.make_async_copy(idx_b.at[k], ib.at[p],
                                      ibsem.at[p]).wait()

                @pl.when(j >= 2)
                def _drain():
                    pltpu.make_async_copy(ra.at[p], zs.at[pl.ds(0, KC), :],
                                          wasem.at[p]).wait()
                    pltpu.make_async_copy(rb.at[p], zd.at[pl.ds(0, KC), :],
                                          wbsem.at[p]).wait()
                ga = pltpu.async_copy(tab_a.at[ia.at[p, 0]], ra.at[p],
                                      gasem.at[p])
                gb = pltpu.async_copy(tab_b.at[ib.at[p, 0]], rb.at[p],
                                      gbsem.at[p])

                @pl.when(k + nw < n_chunks)
                def _pref():
                    fetch_idx(k + nw, 1 - p)
                ga.wait()
                gb.wait()
                base = k * KC
                pltpu.async_copy(ra.at[p], zs.at[pl.ds(base, KC), :],
                                 wasem.at[p])
                pltpu.async_copy(rb.at[p], zd.at[pl.ds(base, KC), :],
                                 wbsem.at[p])
            return carry

        lax.fori_loop(0, per_w, body, 0)
        nu = lax.div(n_chunks - 1 - w, nw) + 1

        @pl.when(w < n_chunks)
        def _ep():
            for t in (2, 1):
                @pl.when(nu >= t)
                def _dr(t=t):
                    p = lax.rem(nu - t, 2)
                    pltpu.make_async_copy(ra.at[p], zs.at[pl.ds(0, KC), :],
                                          wasem.at[p]).wait()
                    pltpu.make_async_copy(rb.at[p], zd.at[pl.ds(0, KC), :],
                                          wbsem.at[p]).wait()

    return gat


def _layer(sum_m, sum_u, cnts, xm_lo, xm_hi, xu_lo, xu_hi,
           wml, bm, wmr, wul, bu, wur, relu, split):
    n = sum_m.shape[1]
    h = wml.shape[0]
    br = 1000
    grid = (n // br,)

    def body(sm, su, cn, xml, xmh, xul, xuh, rwml, rbm, rwmr, rwul, rbu,
             rwur, *outs):
        cm = jnp.maximum(cn[0, :, 0:1], 1.0)
        cu = jnp.maximum(cn[1, :, 0:1], 1.0)

        def sage(slo, shi, cc, xlo, xhi, wl, b, wr):
            m = (jnp.dot(slo / cc, wl[:HALF, :], precision=PREC,
                         preferred_element_type=F32)
                 + jnp.dot(shi / cc, wl[HALF:, :], precision=PREC,
                           preferred_element_type=F32))
            x = (jnp.dot(xlo, wr[:HALF, :], precision=PREC,
                         preferred_element_type=F32)
                 + jnp.dot(xhi, wr[HALF:, :], precision=PREC,
                           preferred_element_type=F32))
            return m + x + b

        hm = sage(sm[0], sm[1], cm, xml[...], xmh[...], rwml[...], rbm[...],
                  rwmr[...])
        hu = sage(su[0], su[1], cu, xul[...], xuh[...], rwul[...], rbu[...],
                  rwur[...])
        if relu:
            hm = jnp.maximum(hm, 0.0)
            hu = jnp.maximum(hu, 0.0)
        if split:
            outs[0][...] = hm[:, :HALF]
            outs[1][...] = hm[:, HALF:]
            outs[2][...] = hu[:, :HALF]
            outs[3][...] = hu[:, HALF:]
        else:
            outs[0][...] = hm
            outs[1][...] = hu

    half_spec = pl.BlockSpec((br, HALF), lambda i: (i, 0))
    w_spec = pl.BlockSpec((h, h), lambda i: (0, 0))
    b_spec = pl.BlockSpec((1, h), lambda i: (0, 0))
    in_specs = [
        pl.BlockSpec((NC, br, HALF), lambda i: (0, i, 0)),
        pl.BlockSpec((NC, br, HALF), lambda i: (0, i, 0)),
        pl.BlockSpec((NC, br, HALF), lambda i: (0, i, 0)),
        half_spec, half_spec, half_spec, half_spec,
        w_spec, b_spec, w_spec, w_spec, b_spec, w_spec,
    ]
    if split:
        out_shape = [jax.ShapeDtypeStruct((n, HALF), F32)] * 4
        out_specs = [pl.BlockSpec((br, HALF), lambda i: (i, 0))] * 4
    else:
        out_shape = [jax.ShapeDtypeStruct((n, h), F32)] * 2
        out_specs = [pl.BlockSpec((br, h), lambda i: (i, 0))] * 2

    return pl.pallas_call(
        body, grid=grid, in_specs=in_specs, out_specs=out_specs,
        out_shape=out_shape,
    )(sum_m, sum_u, cnts, xm_lo, xm_hi, xu_lo, xu_hi,
      wml, bm, wmr, wul, bu, wur)


def _scores(zs, zd, wd2, r):
    n_lab, h = zs.shape
    bl = 1000
    grid = (n_lab // bl,)

    def body(a, b, w, o):
        av = a[...]
        bv = b[...]
        cols = []
        for j in range(r):
            t = jnp.dot(av, w[:, j * h:(j + 1) * h], precision=PREC,
                        preferred_element_type=F32)
            cols.append(jnp.sum(t * bv, axis=1, keepdims=True))
        o[...] = jnp.concatenate(cols, axis=1)

    return pl.pallas_call(
        body, grid=grid,
        in_specs=[pl.BlockSpec((bl, h), lambda i: (i, 0)),
                  pl.BlockSpec((bl, h), lambda i: (i, 0)),
                  pl.BlockSpec((h, r * h), lambda i: (0, 0))],
        out_specs=pl.BlockSpec((bl, r), lambda i: (i, 0)),
        out_shape=jax.ShapeDtypeStruct((n_lab, r), F32),
    )(zs, zd, wd2)


def _log_softmax0(scores):
    def body(x_ref, o_ref):
        x = x_ref[...]
        m = jnp.max(x, axis=0, keepdims=True)
        e = jnp.exp(x - m)
        ssum = jnp.sum(e, axis=0, keepdims=True)
        o_ref[...] = x - m - jnp.log(ssum)

    return pl.pallas_call(
        body, out_shape=jax.ShapeDtypeStruct(scores.shape, F32),
    )(scores)


def kernel(x_user, x_movie, edge_index_um, edge_index_mu, edge_label_index,
           W1_um_l, b1_um, W1_um_r, W1_mu_l, b1_mu, W1_mu_r,
           W2_um_l, b2_um, W2_um_r, W2_mu_l, b2_mu, W2_mu_r, W_dec):
    n, h = x_user.shape
    e = edge_index_um.shape[1]
    n_lab = edge_label_index.shape[1]
    r = W_dec.shape[0]

    su, du = edge_index_um[0], edge_index_um[1]
    sm, dm = edge_index_mu[0], edge_index_mu[1]
    su3 = su.reshape(NS, -1, NB, KC)
    du3 = du.reshape(NS, -1, NB, KC)
    sm3 = sm.reshape(NS, -1, NB, KC)
    dm3 = dm.reshape(NS, -1, NB, KC)
    el0, el1 = edge_label_index[0], edge_label_index[1]
    xu_lo, xu_hi = x_user[:, :HALF], x_user[:, HALF:]
    xm_lo, xm_hi = x_movie[:, :HALF], x_movie[:, HALF:]

    zeros_h = jnp.zeros((n // NS, HALF), F32)
    ones_h = jnp.ones((KC, HALF), F32)

    cnts = _make_counts(n, e)(du3, dm3, zeros_h, ones_h)
    seg = _make_seg_sum(n, e)
    sum_m1, sum_u1 = seg(xu_lo, xu_hi, su3, du3,
                         xm_lo, xm_hi, sm3, dm3, zeros_h)

    hm_lo, hm_hi, hu_lo, hu_hi = _layer(
        sum_m1, sum_u1, cnts, xm_lo, xm_hi, xu_lo, xu_hi,
        W1_um_l, b1_um.reshape(1, -1), W1_um_r,
        W1_mu_l, b1_mu.reshape(1, -1), W1_mu_r, relu=True, split=True)

    sum_m2, sum_u2 = seg(hu_lo, hu_hi, su3, du3,
                         hm_lo, hm_hi, sm3, dm3, zeros_h)

    z_movie, z_user = _layer(
        sum_m2, sum_u2, cnts, hm_lo, hm_hi, hu_lo, hu_hi,
        W2_um_l, b2_um.reshape(1, -1), W2_um_r,
        W2_mu_l, b2_mu.reshape(1, -1), W2_mu_r, relu=False, split=False)

    zs, zd = _make_pair_gather(n_lab, h)(z_user, z_movie,
                                         el0.reshape(-1, 1, KC),
                                         el1.reshape(-1, 1, KC))

    wd2 = jnp.transpose(W_dec, (1, 0, 2)).reshape(h, r * h)
    sc = _scores(zs, zd, wd2, r)
    return _log_softmax0(sc)


# speedup vs baseline: 1.0096x; 1.0051x over previous
import functools

import jax
import jax.numpy as jnp
from jax import lax
from jax.experimental import pallas as pl
from jax.experimental.pallas import tpu as pltpu
from jax.experimental.pallas import tpu_sc as plsc

NC = 2
NS = 16
HALF = 128
KC = 40
PREC = None
NB = 5
F32 = jnp.float32


def _mesh():
    return plsc.VectorSubcoreMesh(core_axis_name="c", subcore_axis_name="s",
                                  num_cores=NC, num_subcores=NS)


def _make_seg_sum(n_dst, n_edges):
    e_per_sub = n_edges // NS
    n_chunks = e_per_sub // KC
    n_groups = n_chunks // NB
    assert n_chunks % NB == 0
    slab = (n_dst // NS) & ~7
    tail = n_dst - NS * slab

    @functools.partial(
        pl.kernel,
        out_type=jax.ShapeDtypeStruct((NC, n_dst, HALF), F32),
        mesh=_mesh(),
        scratch_types=[
            pltpu.VMEM((2, NB, KC), jnp.int32),
            pltpu.VMEM((2, NB, KC), jnp.int32),
            pltpu.VMEM((NB, KC, HALF), F32),
            pltpu.VMEM_SHARED((n_dst, HALF), F32),
            pltpu.SemaphoreType.DMA((2,)),
            pltpu.SemaphoreType.DMA((2,)),
            pltpu.SemaphoreType.DMA((NB,)),
            pltpu.SemaphoreType.DMA((NB,)),
        ],
    )
    def seg(x_lo, x_hi, src, dst, zrs, out, src_b, dst_b, rows_v, acc,
            s_isem, d_isem, gsem, ssem):
        c = lax.axis_index("c")
        s = lax.axis_index("s")
        r0 = s * slab

        def fetch_idx(g, p):
            pltpu.async_copy(src.at[s, g], src_b.at[p], s_isem.at[p])
            pltpu.async_copy(dst.at[s, g], dst_b.at[p], d_isem.at[p])

        def wait_idx(g, p):
            pltpu.make_async_copy(src.at[s, g], src_b.at[p],
                                  s_isem.at[p]).wait()
            pltpu.make_async_copy(dst.at[s, g], dst_b.at[p],
                                  d_isem.at[p]).wait()

        fetch_idx(0, 0)
        pltpu.sync_copy(zrs.at[pl.ds(0, slab)], acc.at[pl.ds(r0, slab)])
        if tail:
            @pl.when(s == NS - 1)
            def _zt():
                pltpu.sync_copy(zrs.at[pl.ds(0, tail)],
                                acc.at[pl.ds(NS * slab, tail)])
        plsc.subcore_barrier()

        def run(x_tab):
            def group(g, carry):
                p = lax.rem(g, 2)
                wait_idx(g, p)
                descs = []
                for b in range(NB):
                    @pl.when(g > 0)
                    def _drain(b=b):
                        pltpu.make_async_copy(
                            rows_v.at[b], acc.at[dst_b.at[1 - p, b]],
                            ssem.at[b]).wait()
                    descs.append(pltpu.async_copy(
                        x_tab.at[src_b.at[p, b]], rows_v.at[b], gsem.at[b]))

                @pl.when(g + 1 < n_groups)
                def _pref():
                    fetch_idx(g + 1, 1 - p)
                for b in range(NB):
                    descs[b].wait()
                    pltpu.async_copy(rows_v.at[b], acc.at[dst_b.at[p, b]],
                                     ssem.at[b], add=True)
                return carry
            lax.fori_loop(0, n_groups, group, 0)
            pl_ = (n_groups - 1) % 2
            for b in range(NB):
                pltpu.make_async_copy(rows_v.at[b], acc.at[dst_b.at[pl_, b]],
                                      ssem.at[b]).wait()

        @pl.when(c == 0)
        def _lo():
            run(x_lo)

        @pl.when(c == 1)
        def _hi():
            run(x_hi)

        plsc.subcore_barrier()
        pltpu.sync_copy(acc.at[pl.ds(r0, slab)],
                        out.at[c, pl.ds(r0, slab), :])
        if tail:
            @pl.when(s == NS - 1)
            def _ft():
                pltpu.sync_copy(acc.at[pl.ds(NS * slab, tail)],
                                out.at[c, pl.ds(NS * slab, tail), :])

    return seg


def _make_counts(n_dst, n_edges):
    e_per_sub = n_edges // NS
    n_chunks = e_per_sub // KC
    slab = (n_dst // NS) & ~7
    tail = n_dst - NS * slab

    n_groups = n_chunks // NB

    @functools.partial(
        pl.kernel,
        out_type=jax.ShapeDtypeStruct((NC, n_dst, HALF), F32),
        mesh=_mesh(),
        scratch_types=[
            pltpu.VMEM((2, NB, KC), jnp.int32),
            pltpu.VMEM((KC, HALF), F32),
            pltpu.VMEM_SHARED((n_dst, HALF), F32),
            pltpu.SemaphoreType.DMA((2,)),
            pltpu.SemaphoreType.DMA((NB,)),
        ],
    )
    def cnt(dst_a, dst_b, zrs, ones, out, idx_b, ones_v, acc, isem, ssem):
        c = lax.axis_index("c")
        s = lax.axis_index("s")
        r0 = s * slab
        pltpu.sync_copy(zrs.at[pl.ds(0, slab)], acc.at[pl.ds(r0, slab)])
        if tail:
            @pl.when(s == NS - 1)
            def _zt():
                pltpu.sync_copy(zrs.at[pl.ds(0, tail)],
                                acc.at[pl.ds(NS * slab, tail)])
        pltpu.sync_copy(ones, ones_v)

        def run(dref):
            def fetch_idx(g, p):
                pltpu.async_copy(dref.at[s, g], idx_b.at[p], isem.at[p])

            def wait_idx(g, p):
                pltpu.make_async_copy(dref.at[s, g], idx_b.at[p],
                                      isem.at[p]).wait()

            fetch_idx(0, 0)
            plsc.subcore_barrier()

            def group(g, carry):
                p = lax.rem(g, 2)
                wait_idx(g, p)
                for b in range(NB):
                    @pl.when(g > 0)
                    def _drain(b=b):
                        pltpu.make_async_copy(
                            ones_v, acc.at[idx_b.at[1 - p, b]],
                            ssem.at[b]).wait()
                    pltpu.async_copy(ones_v, acc.at[idx_b.at[p, b]],
                                     ssem.at[b], add=True)

                @pl.when(g + 1 < n_groups)
                def _pref():
                    fetch_idx(g + 1, 1 - p)
                return carry
            lax.fori_loop(0, n_groups, group, 0)
            pl_ = (n_groups - 1) % 2
            for b in range(NB):
                pltpu.make_async_copy(ones_v, acc.at[idx_b.at[pl_, b]],
                                      ssem.at[b]).wait()

        @pl.when(c == 0)
        def _a():
            run(dst_a)

        @pl.when(c == 1)
        def _b():
            run(dst_b)

        plsc.subcore_barrier()
        pltpu.sync_copy(acc.at[pl.ds(r0, slab)],
                        out.at[c, pl.ds(r0, slab), :])
        if tail:
            @pl.when(s == NS - 1)
            def _ft():
                pltpu.sync_copy(acc.at[pl.ds(NS * slab, tail)],
                                out.at[c, pl.ds(NS * slab, tail), :])

    return cnt


def _make_pair_gather(n_lab, width):
    n_chunks = n_lab // KC
    nw = NC * NS
    per_w = -(-n_chunks // nw)

    @functools.partial(
        pl.kernel,
        out_type=(jax.ShapeDtypeStruct((n_lab, width), F32),
                  jax.ShapeDtypeStruct((n_lab, width), F32)),
        mesh=_mesh(),
        scratch_types=[
            pltpu.VMEM((2, 1, KC), jnp.int32),
            pltpu.VMEM((2, 1, KC), jnp.int32),
            pltpu.VMEM((2, KC, width), F32),
            pltpu.VMEM((2, KC, width), F32),
            pltpu.SemaphoreType.DMA((2,)),
            pltpu.SemaphoreType.DMA((2,)),
            pltpu.SemaphoreType.DMA((2,)),
            pltpu.SemaphoreType.DMA((2,)),
            pltpu.SemaphoreType.DMA((2,)),
            pltpu.SemaphoreType.DMA((2,)),
        ],
    )
    def gat(tab_a, tab_b, idx_a, idx_b, zs, zd, ia, ib, ra, rb,
            iasem, ibsem, gasem, gbsem, wasem, wbsem):
        c = lax.axis_index("c")
        s = lax.axis_index("s")
        w = s * NC + c

        def fetch_idx(k, p):
            pltpu.async_copy(idx_a.at[k], ia.at[p], iasem.at[p])
            pltpu.async_copy(idx_b.at[k], ib.at[p], ibsem.at[p])

        @pl.when(w < n_chunks)
        def _p0():
            fetch_idx(w, 0)

        def body(j, carry):
            p = lax.rem(j, 2)
            k = w + nw * j

            @pl.when(k < n_chunks)
            def _work():
                pltpu.make_async_copy(idx_a.at[k], ia.at[p],
                                      iasem.at[p]).wait()
                pltpu.make_async_copy(idx_b.at[k], ib.at[p],
                                      ibsem.at[p]).wait()

                @pl.when(j >= 2)
                def _drain():
                    pltpu.make_async_copy(ra.at[p], zs.at[pl.ds(0, KC), :],
                                          wasem.at[p]).wait()
                    pltpu.make_async_copy(rb.at[p], zd.at[pl.ds(0, KC), :],
                                          wbsem.at[p]).wait()
                ga = pltpu.async_copy(tab_a.at[ia.at[p, 0]], ra.at[p],
                                      gasem.at[p])
                gb = pltpu.async_copy(tab_b.at[ib.at[p, 0]], rb.at[p],
                                      gbsem.at[p])

                @pl.when(k + nw < n_chunks)
                def _pref():
                    fetch_idx(k + nw, 1 - p)
                ga.wait()
                gb.wait()
                base = k * KC
                pltpu.async_copy(ra.at[p], zs.at[pl.ds(base, KC), :],
                                 wasem.at[p])
                pltpu.async_copy(rb.at[p], zd.at[pl.ds(base, KC), :],
                                 wbsem.at[p])
            return carry

        lax.fori_loop(0, per_w, body, 0)
        nu = lax.div(n_chunks - 1 - w, nw) + 1

        @pl.when(w < n_chunks)
        def _ep():
            for t in (2, 1):
                @pl.when(nu >= t)
                def _dr(t=t):
                    p = lax.rem(nu - t, 2)
                    pltpu.make_async_copy(ra.at[p], zs.at[pl.ds(0, KC), :],
                                          wasem.at[p]).wait()
                    pltpu.make_async_copy(rb.at[p], zd.at[pl.ds(0, KC), :],
                                          wbsem.at[p]).wait()

    return gat


def _layer(sum_m, sum_u, cnts, xm_lo, xm_hi, xu_lo, xu_hi,
           wml, bm, wmr, wul, bu, wur, relu, split):
    n = sum_m.shape[1]
    h = wml.shape[0]
    br = 1000
    grid = (n // br,)

    def body(sm, su, cn, xml, xmh, xul, xuh, rwml, rbm, rwmr, rwul, rbu,
             rwur, *outs):
        cm = jnp.maximum(cn[0, :, 0:1], 1.0)
        cu = jnp.maximum(cn[1, :, 0:1], 1.0)

        def sage(slo, shi, cc, xlo, xhi, wl, b, wr):
            m = (jnp.dot(slo / cc, wl[:HALF, :], precision=PREC,
                         preferred_element_type=F32)
                 + jnp.dot(shi / cc, wl[HALF:, :], precision=PREC,
                           preferred_element_type=F32))
            x = (jnp.dot(xlo, wr[:HALF, :], precision=PREC,
                         preferred_element_type=F32)
                 + jnp.dot(xhi, wr[HALF:, :], precision=PREC,
                           preferred_element_type=F32))
            return m + x + b

        hm = sage(sm[0], sm[1], cm, xml[...], xmh[...], rwml[...], rbm[...],
                  rwmr[...])
        hu = sage(su[0], su[1], cu, xul[...], xuh[...], rwul[...], rbu[...],
                  rwur[...])
        if relu:
            hm = jnp.maximum(hm, 0.0)
            hu = jnp.maximum(hu, 0.0)
        if split:
            outs[0][...] = hm[:, :HALF]
            outs[1][...] = hm[:, HALF:]
            outs[2][...] = hu[:, :HALF]
            outs[3][...] = hu[:, HALF:]
        else:
            outs[0][...] = hm
            outs[1][...] = hu

    half_spec = pl.BlockSpec((br, HALF), lambda i: (i, 0))
    w_spec = pl.BlockSpec((h, h), lambda i: (0, 0))
    b_spec = pl.BlockSpec((1, h), lambda i: (0, 0))
    in_specs = [
        pl.BlockSpec((NC, br, HALF), lambda i: (0, i, 0)),
        pl.BlockSpec((NC, br, HALF), lambda i: (0, i, 0)),
        pl.BlockSpec((NC, br, HALF), lambda i: (0, i, 0)),
        half_spec, half_spec, half_spec, half_spec,
        w_spec, b_spec, w_spec, w_spec, b_spec, w_spec,
    ]
    if split:
        out_shape = [jax.ShapeDtypeStruct((n, HALF), F32)] * 4
        out_specs = [pl.BlockSpec((br, HALF), lambda i: (i, 0))] * 4
    else:
        out_shape = [jax.ShapeDtypeStruct((n, h), F32)] * 2
        out_specs = [pl.BlockSpec((br, h), lambda i: (i, 0))] * 2

    return pl.pallas_call(
        body, grid=grid, in_specs=in_specs, out_specs=out_specs,
        out_shape=out_shape,
    )(sum_m, sum_u, cnts, xm_lo, xm_hi, xu_lo, xu_hi,
      wml, bm, wmr, wul, bu, wur)


def _scores(zs, zd, wd2, r):
    n_lab, h = zs.shape
    bl = 1000
    grid = (n_lab // bl,)

    def body(a, b, w, o):
        av = a[...]
        bv = b[...]
        cols = []
        for j in range(r):
            t = jnp.dot(av, w[:, j * h:(j + 1) * h], precision=PREC,
                        preferred_element_type=F32)
            cols.append(jnp.sum(t * bv, axis=1, keepdims=True))
        o[...] = jnp.concatenate(cols, axis=1)

    return pl.pallas_call(
        body, grid=grid,
        in_specs=[pl.BlockSpec((bl, h), lambda i: (i, 0)),
                  pl.BlockSpec((bl, h), lambda i: (i, 0)),
                  pl.BlockSpec((h, r * h), lambda i: (0, 0))],
        out_specs=pl.BlockSpec((bl, r), lambda i: (i, 0)),
        out_shape=jax.ShapeDtypeStruct((n_lab, r), F32),
    )(zs, zd, wd2)


def _log_softmax0(scores):
    def body(x_ref, o_ref):
        x = x_ref[...]
        m = jnp.max(x, axis=0, keepdims=True)
        e = jnp.exp(x - m)
        ssum = jnp.sum(e, axis=0, keepdims=True)
        o_ref[...] = x - m - jnp.log(ssum)

    return pl.pallas_call(
        body, out_shape=jax.ShapeDtypeStruct(scores.shape, F32),
    )(scores)


def kernel(x_user, x_movie, edge_index_um, edge_index_mu, edge_label_index,
           W1_um_l, b1_um, W1_um_r, W1_mu_l, b1_mu, W1_mu_r,
           W2_um_l, b2_um, W2_um_r, W2_mu_l, b2_mu, W2_mu_r, W_dec):
    n, h = x_user.shape
    e = edge_index_um.shape[1]
    n_lab = edge_label_index.shape[1]
    r = W_dec.shape[0]

    su, du = edge_index_um[0], edge_index_um[1]
    sm, dm = edge_index_mu[0], edge_index_mu[1]
    su3 = su.reshape(NS, -1, NB, KC)
    du3 = du.reshape(NS, -1, NB, KC)
    sm3 = sm.reshape(NS, -1, NB, KC)
    dm3 = dm.reshape(NS, -1, NB, KC)
    el0, el1 = edge_label_index[0], edge_label_index[1]
    xu_lo, xu_hi = x_user[:, :HALF], x_user[:, HALF:]
    xm_lo, xm_hi = x_movie[:, :HALF], x_movie[:, HALF:]

    zeros_h = jnp.zeros((n // NS, HALF), F32)
    ones_h = jnp.ones((KC, HALF), F32)

    cnts = _make_counts(n, e)(du3, dm3, zeros_h, ones_h)
    seg = _make_seg_sum(n, e)
    sum_m1 = seg(xu_lo, xu_hi, su3, du3, zeros_h)
    sum_u1 = seg(xm_lo, xm_hi, sm3, dm3, zeros_h)

    hm_lo, hm_hi, hu_lo, hu_hi = _layer(
        sum_m1, sum_u1, cnts, xm_lo, xm_hi, xu_lo, xu_hi,
        W1_um_l, b1_um.reshape(1, -1), W1_um_r,
        W1_mu_l, b1_mu.reshape(1, -1), W1_mu_r, relu=True, split=True)

    sum_m2 = seg(hu_lo, hu_hi, su3, du3, zeros_h)
    sum_u2 = seg(hm_lo, hm_hi, sm3, dm3, zeros_h)

    z_movie, z_user = _layer(
        sum_m2, sum_u2, cnts, hm_lo, hm_hi, hu_lo, hu_hi,
        W2_um_l, b2_um.reshape(1, -1), W2_um_r,
        W2_mu_l, b2_mu.reshape(1, -1), W2_mu_r, relu=False, split=False)

    zs, zd = _make_pair_gather(n_lab, h)(z_user, z_movie,
                                         el0.reshape(-1, 1, KC),
                                         el1.reshape(-1, 1, KC))

    wd2 = jnp.transpose(W_dec, (1, 0, 2)).reshape(h, r * h)
    sc = _scores(zs, zd, wd2, r)
    return _log_softmax0(sc)
